# baseline (speedup 1.0000x reference)
import functools

import jax
import jax.numpy as jnp
from jax import lax
from jax.experimental import pallas as pl
from jax.experimental.pallas import tpu as pltpu
from jax.experimental.pallas import tpu_sc as plsc

C = 4096
NC, NS, L = 2, 16, 16
NW = NC * NS
W = C // NW
NG = W // L
R = 128
NB = C // R


def _body(act_hbm, hold_hbm, dom_hbm, outdom_hbm, outhold_hbm,
          act_v, hold_v, nh_v, dbuf, obuf,
          sem_in0, sem_in1, sem_out0, sem_out1):
    wid = lax.axis_index("s") * NC + lax.axis_index("c")
    n0 = wid * W
    sem_in = (sem_in0, sem_in1)
    sem_out = (sem_out0, sem_out1)

    pltpu.sync_copy(act_hbm.at[pl.ds(n0, W)], act_v)
    pltpu.sync_copy(hold_hbm.at[:], hold_v.at[pl.ds(0, C)])

    a_g = [act_v[pl.ds(g * L, L)] for g in range(NG)]
    A_g = [1.0 - a for a in a_g]

    zero = jnp.zeros((L,), jnp.float32)
    carry = tuple(zero for _ in range(3 * NG))

    def in_copy(j, p):
        return pltpu.async_copy(
            dom_hbm.at[pl.ds(j * R, R), pl.ds(n0, W)], dbuf.at[p], sem_in[p])

    def out_copy(j, p):
        return pltpu.async_copy(
            obuf.at[p], outdom_hbm.at[pl.ds(j * R, R), pl.ds(n0, W)],
            sem_out[p])

    in_copy(0, 0)
    in_copy(1, 1)

    RU = 8
    NCH = R // RU

    def pair_body(i, carry):
        for p in (0, 1):
            j = 2 * i + p
            db = dbuf.at[p]
            ob = obuf.at[p]
            pltpu.make_async_copy(
                dom_hbm.at[pl.ds(0, R), pl.ds(n0, W)], db, sem_in[p]).wait()

            @pl.when(i >= 1)
            def _():
                pltpu.make_async_copy(
                    ob, outdom_hbm.at[pl.ds(0, R), pl.ds(n0, W)],
                    sem_out[p]).wait()

            m0 = j * R

            def chunk_body(cc, t, db=db, ob=ob, m0=m0):
                mb = cc * RU
                hvec = hold_v[pl.ds(m0 + mb, L)]
                t = list(t)
                for k in range(RU):
                    hv = jnp.full((L,), hvec[k], jnp.float32)
                    Hv = 1.0 - hv
                    for g in range(NG):
                        d = db[mb + k, pl.ds(g * L, L)]
                        p1 = d * A_g[g]
                        pr = d * hv
                        p2 = d - pr
                        ob[mb + k, pl.ds(g * L, L)] = p1 + p2 - p1 * p2
                        t0, t1, t2 = t[3 * g], t[3 * g + 1], t[3 * g + 2]
                        n0v = jnp.maximum(t0, pr)
                        r1 = jnp.minimum(t0, pr)
                        n1v = jnp.maximum(t1, r1)
                        r2 = jnp.minimum(t1, r1)
                        n2v = jnp.maximum(t2, r2)
                        t[3 * g], t[3 * g + 1], t[3 * g + 2] = n0v, n1v, n2v
                return tuple(t)

            carry = lax.fori_loop(0, NCH, chunk_body, tuple(carry))
            out_copy(j, p)

            @pl.when(j + 2 < NB)
            def _():
                in_copy(j + 2, p)

        return carry

    carry = lax.fori_loop(0, NB // 2, pair_body, carry)

    for p in (0, 1):
        pltpu.make_async_copy(
            obuf.at[p], outdom_hbm.at[pl.ds(0, R), pl.ds(n0, W)],
            sem_out[p]).wait()

    for g in range(NG):
        v0 = carry[3 * g] * a_g[g]
        v1 = carry[3 * g + 1] * a_g[g]
        v2 = carry[3 * g + 2] * a_g[g]
        nh_v[pl.ds(g * L, L)] = 1.0 - (1.0 - v0) * (1.0 - v1) * (1.0 - v2)
    pltpu.sync_copy(nh_v, outhold_hbm.at[pl.ds(n0, W)])


_sc_call = functools.partial(
    pl.kernel,
    out_type=[
        jax.ShapeDtypeStruct((C, C), jnp.float32),
        jax.ShapeDtypeStruct((C,), jnp.float32),
    ],
    mesh=plsc.VectorSubcoreMesh(
        core_axis_name="c", subcore_axis_name="s", num_cores=NC,
        num_subcores=NS),
    compiler_params=pltpu.CompilerParams(use_tc_tiling_on_sc=True),
    scratch_types=[
        pltpu.VMEM((W,), jnp.float32),
        pltpu.VMEM((C + L,), jnp.float32),
        pltpu.VMEM((W,), jnp.float32),
        pltpu.VMEM((2, R, W), jnp.float32),
        pltpu.VMEM((2, R, W), jnp.float32),
        pltpu.SemaphoreType.DMA,
        pltpu.SemaphoreType.DMA,
        pltpu.SemaphoreType.DMA,
        pltpu.SemaphoreType.DMA,
    ],
)(_body)


def kernel(action, holding, dominos):
    dom = dominos.reshape(C, C)
    out_dom, out_hold = _sc_call(action, holding, dom)
    return out_hold, out_dom.reshape(-1)


# speedup vs baseline: 2.0346x; 2.0346x over previous
import functools

import jax
import jax.numpy as jnp
from jax import lax
from jax.experimental import pallas as pl
from jax.experimental.pallas import tpu as pltpu
from jax.experimental.pallas import tpu_sc as plsc

C = 4096
NC, NS, L = 2, 16, 16
NW = NC * NS
W = C // NW
NG = W // L
R = 128
NB = C // R


def _body(act_hbm, hold_hbm, dom_hbm, outdom_hbm, outhold_hbm,
          act_v, hold_v, nh_v, dbuf, obuf,
          sem_in0, sem_in1, sem_out0, sem_out1):
    wid = lax.axis_index("s") * NC + lax.axis_index("c")
    n0 = wid * W
    sem_in = (sem_in0, sem_in1)
    sem_out = (sem_out0, sem_out1)

    pltpu.sync_copy(act_hbm.at[pl.ds(n0, W)], act_v)
    pltpu.sync_copy(hold_hbm.at[:], hold_v.at[pl.ds(0, C)])

    a_g = [act_v[pl.ds(g * L, L)] for g in range(NG)]
    A_g = [1.0 - a for a in a_g]

    zero = jnp.zeros((L,), jnp.float32)
    carry = tuple(zero for _ in range(3 * NG))

    def in_copy(j, p):
        return pltpu.async_copy(
            dom_hbm.at[pl.ds(j * R, R), wid], dbuf.at[p], sem_in[p])

    def out_copy(j, p):
        return pltpu.async_copy(
            obuf.at[p], outdom_hbm.at[pl.ds(j * R, R), wid], sem_out[p])

    in_copy(0, 0)
    in_copy(1, 1)

    RU = 8
    NCH = R // RU

    def pair_body(i, carry):
        for p in (0, 1):
            j = 2 * i + p
            db = dbuf.at[p]
            ob = obuf.at[p]
            pltpu.make_async_copy(
                dom_hbm.at[pl.ds(0, R), wid], db, sem_in[p]).wait()

            @pl.when(i >= 1)
            def _():
                pltpu.make_async_copy(
                    ob, outdom_hbm.at[pl.ds(0, R), wid], sem_out[p]).wait()

            m0 = j * R

            def chunk_body(cc, t, db=db, ob=ob, m0=m0):
                mb = cc * RU
                hvec = hold_v[pl.ds(m0 + mb, L)]
                t = list(t)
                for k in range(RU):
                    hv = jnp.full((L,), hvec[k], jnp.float32)
                    Hv = 1.0 - hv
                    for g in range(NG):
                        d = db[mb + k, pl.ds(g * L, L)]
                        p1 = d * A_g[g]
                        pr = d * hv
                        p2 = d - pr
                        ob[mb + k, pl.ds(g * L, L)] = p1 + p2 - p1 * p2
                        t0, t1, t2 = t[3 * g], t[3 * g + 1], t[3 * g + 2]
                        n0v = jnp.maximum(t0, pr)
                        r1 = jnp.minimum(t0, pr)
                        n1v = jnp.maximum(t1, r1)
                        r2 = jnp.minimum(t1, r1)
                        n2v = jnp.maximum(t2, r2)
                        t[3 * g], t[3 * g + 1], t[3 * g + 2] = n0v, n1v, n2v
                return tuple(t)

            carry = lax.fori_loop(0, NCH, chunk_body, tuple(carry))
            out_copy(j, p)

            @pl.when(j + 2 < NB)
            def _():
                in_copy(j + 2, p)

        return carry

    carry = lax.fori_loop(0, NB // 2, pair_body, carry)

    for p in (0, 1):
        pltpu.make_async_copy(
            obuf.at[p], outdom_hbm.at[pl.ds(0, R), wid], sem_out[p]).wait()

    for g in range(NG):
        v0 = carry[3 * g] * a_g[g]
        v1 = carry[3 * g + 1] * a_g[g]
        v2 = carry[3 * g + 2] * a_g[g]
        nh_v[pl.ds(g * L, L)] = 1.0 - (1.0 - v0) * (1.0 - v1) * (1.0 - v2)
    pltpu.sync_copy(nh_v, outhold_hbm.at[pl.ds(n0, W)])


_sc_call = functools.partial(
    pl.kernel,
    out_type=[
        jax.ShapeDtypeStruct((C, NW, W), jnp.float32),
        jax.ShapeDtypeStruct((C,), jnp.float32),
    ],
    mesh=plsc.VectorSubcoreMesh(
        core_axis_name="c", subcore_axis_name="s", num_cores=NC,
        num_subcores=NS),
    compiler_params=pltpu.CompilerParams(use_tc_tiling_on_sc=True),
    scratch_types=[
        pltpu.VMEM((W,), jnp.float32),
        pltpu.VMEM((C + L,), jnp.float32),
        pltpu.VMEM((W,), jnp.float32),
        pltpu.VMEM((2, R, W), jnp.float32),
        pltpu.VMEM((2, R, W), jnp.float32),
        pltpu.SemaphoreType.DMA,
        pltpu.SemaphoreType.DMA,
        pltpu.SemaphoreType.DMA,
        pltpu.SemaphoreType.DMA,
    ],
)(_body)


def kernel(action, holding, dominos):
    dom = dominos.reshape(C, NW, W)
    out_dom, out_hold = _sc_call(action, holding, dom)
    return out_hold, out_dom.reshape(-1)


# speedup vs baseline: 2.7118x; 1.3328x over previous
import functools

import jax
import jax.numpy as jnp
from jax import lax
from jax.experimental import pallas as pl
from jax.experimental.pallas import tpu as pltpu
from jax.experimental.pallas import tpu_sc as plsc

C = 4096
NC, NS, L = 2, 16, 16
NW = NC * NS
W = C // NW
NG = W // L
R = 128
NB = C // R


def _sc_body(act_hbm, hold_hbm, dom_hbm, outhold_hbm,
             act_v, hold_v, nh_v, dbuf, sem_in0, sem_in1):
    wid = lax.axis_index("s") * NC + lax.axis_index("c")
    sem_in = (sem_in0, sem_in1)
    n0 = wid * W

    pltpu.sync_copy(act_hbm.at[pl.ds(n0, W)], act_v)
    pltpu.sync_copy(hold_hbm.at[:], hold_v.at[pl.ds(0, C)])

    a_g = [act_v[pl.ds(g * L, L)] for g in range(NG)]

    zero = jnp.zeros((L,), jnp.float32)
    carry = tuple(zero for _ in range(3 * NG))

    def in_copy(j, p):
        return pltpu.async_copy(
            dom_hbm.at[pl.ds(j * R, R), wid], dbuf.at[p], sem_in[p])

    in_copy(0, 0)
    in_copy(1, 1)

    RU = 8
    NCH = R // RU

    def pair_body(i, carry):
        for p in (0, 1):
            j = 2 * i + p
            db = dbuf.at[p]
            pltpu.make_async_copy(
                dom_hbm.at[pl.ds(0, R), wid], db, sem_in[p]).wait()
            m0 = j * R

            def chunk_body(cc, t, db=db, m0=m0):
                mb = cc * RU
                hvec = hold_v[pl.ds(m0 + mb, L)]
                t = list(t)
                for k in range(RU):
                    hv = jnp.full((L,), hvec[k], jnp.float32)
                    for g in range(NG):
                        d = db[mb + k, pl.ds(g * L, L)]
                        pr = d * hv
                        t0, t1, t2 = t[3 * g], t[3 * g + 1], t[3 * g + 2]
                        n0v = jnp.maximum(t0, pr)
                        r1 = jnp.minimum(t0, pr)
                        n1v = jnp.maximum(t1, r1)
                        r2 = jnp.minimum(t1, r1)
                        n2v = jnp.maximum(t2, r2)
                        t[3 * g], t[3 * g + 1], t[3 * g + 2] = n0v, n1v, n2v
                return tuple(t)

            carry = lax.fori_loop(0, NCH, chunk_body, tuple(carry))

            @pl.when(j + 2 < NB)
            def _():
                in_copy(j + 2, p)

        return carry

    carry = lax.fori_loop(0, NB // 2, pair_body, carry)

    for g in range(NG):
        v0 = carry[3 * g] * a_g[g]
        v1 = carry[3 * g + 1] * a_g[g]
        v2 = carry[3 * g + 2] * a_g[g]
        nh_v[pl.ds(g * L, L)] = 1.0 - (1.0 - v0) * (1.0 - v1) * (1.0 - v2)
    pltpu.sync_copy(nh_v, outhold_hbm.at[pl.ds(n0, W)])


_sc_call = functools.partial(
    pl.kernel,
    out_type=jax.ShapeDtypeStruct((C,), jnp.float32),
    mesh=plsc.VectorSubcoreMesh(
        core_axis_name="c", subcore_axis_name="s", num_cores=NC,
        num_subcores=NS),
    scratch_types=[
        pltpu.VMEM((W,), jnp.float32),
        pltpu.VMEM((C + L,), jnp.float32),
        pltpu.VMEM((W,), jnp.float32),
        pltpu.VMEM((2, R, W), jnp.float32),
        pltpu.SemaphoreType.DMA,
        pltpu.SemaphoreType.DMA,
    ],
)(_sc_body)


TBM = 128


def _tc_body(hold_smem, act_ref, dom_ref, out_ref):
    i0 = pl.program_id(0) * TBM
    A = 1.0 - act_ref[...]

    def slab(i, _):
        h = hold_smem[i0 + i]
        d = dom_ref[i]
        p1 = d * A
        p2 = d * (1.0 - h)
        out_ref[i] = p1 + p2 - p1 * p2
        return 0

    lax.fori_loop(0, TBM, slab, 0)


_tc_call = pl.pallas_call(
    _tc_body,
    grid=(C // TBM,),
    in_specs=[
        pl.BlockSpec(memory_space=pltpu.SMEM),
        pl.BlockSpec((NW, W), lambda i: (0, 0)),
        pl.BlockSpec((TBM, NW, W), lambda i: (i, 0, 0)),
    ],
    out_specs=pl.BlockSpec((TBM, NW, W), lambda i: (i, 0, 0)),
    out_shape=jax.ShapeDtypeStruct((C, NW, W), jnp.float32),
)


def kernel(action, holding, dominos):
    dom = dominos.reshape(C, NW, W)
    act2 = action.reshape(NW, W)
    out_hold = _sc_call(action, holding, dom)
    out_dom = _tc_call(holding, act2, dom)
    return out_hold, out_dom.reshape(-1)
